# baseline (speedup 1.0000x reference)
import functools

import jax
import jax.numpy as jnp
from jax import lax
from jax.experimental import pallas as pl
from jax.experimental.pallas import tpu as pltpu
from jax.experimental.pallas import tpu_sc as plsc

NC = 2
NS = 16
NW = NC * NS
LANES = 16
D = 256
NV = D // LANES
CH = 64


def _sc_bind_sum(nch):
    mesh = plsc.VectorSubcoreMesh(core_axis_name="c", subcore_axis_name="s")

    def body(tab, ia, ib, out, ia_v, ib_v, b0, b1, acc_v, sem0, sem1):
        wid = lax.axis_index("s") * NC + lax.axis_index("c")
        pltpu.sync_copy(ia.at[wid], ia_v)
        pltpu.sync_copy(ib.at[wid], ib_v)
        sems = [sem0, sem1]

        def fire(c, par):
            pltpu.async_copy(tab.at[ia_v.at[c]], b0.at[par], sems[par])
            pltpu.async_copy(tab.at[ib_v.at[c]], b1.at[par], sems[par])

        def drain(c, par):
            pltpu.make_async_copy(tab.at[ia_v.at[c]], b0.at[par], sems[par]).wait()
            pltpu.make_async_copy(tab.at[ib_v.at[c]], b1.at[par], sems[par]).wait()

        fire(0, 0)

        zero = jnp.zeros((LANES,), jnp.float32)
        accs0 = (zero,) * NV

        def pair_body(i, accs):
            for par in range(2):
                c = 2 * i + par
                drain(c, par)

                @pl.when(c + 1 < nch)
                def _():
                    fire(c + 1, 1 - par)

                def edge_body(e, a):
                    return tuple(
                        a[v]
                        + b0[par, e, pl.ds(LANES * v, LANES)]
                        * b1[par, e, pl.ds(LANES * v, LANES)]
                        for v in range(NV)
                    )

                accs = lax.fori_loop(0, CH, edge_body, accs)
            return accs

        accs = lax.fori_loop(0, nch // 2, pair_body, accs0)
        for v in range(NV):
            acc_v[pl.ds(LANES * v, LANES)] = accs[v]
        pltpu.sync_copy(acc_v, out.at[wid])

    return pl.kernel(
        body,
        out_type=jax.ShapeDtypeStruct((NW, D), jnp.float32),
        mesh=mesh,
        scratch_types=[
            pltpu.VMEM((nch, CH), jnp.int32),
            pltpu.VMEM((nch, CH), jnp.int32),
            pltpu.VMEM((2, CH, D), jnp.float32),
            pltpu.VMEM((2, CH, D), jnp.float32),
            pltpu.VMEM((D,), jnp.float32),
            pltpu.SemaphoreType.DMA,
            pltpu.SemaphoreType.DMA,
        ],
    )


def _tc_reduce_am(part_ref, am_ref, out_ref):
    enc = jnp.sum(part_ref[...], axis=0, keepdims=True)
    out_ref[...] = lax.dot_general(
        enc, am_ref[...], (((1,), (1,)), ((), ())),
        preferred_element_type=jnp.float32,
    )


def kernel(x, edge_index, pr, ids_weight, am_weight):
    n = x.shape[0]
    d = ids_weight.shape[1]
    e = edge_index.shape[1]

    pr_argsort = jnp.argsort(pr)
    table = (
        jnp.zeros((n + 8, d), jnp.float32)
        .at[pr_argsort]
        .set(ids_weight[:n])
    )

    a = jnp.minimum(edge_index[0], edge_index[1])
    b = jnp.maximum(edge_index[0], edge_index[1])
    keys = a * n + b
    ks, sa, sb = lax.sort((keys, a, b), num_keys=1)
    first = jnp.concatenate(
        [jnp.ones((1,), dtype=bool), ks[1:] != ks[:-1]]
    )
    zrow = jnp.int32(n)
    ia = jnp.where(first, sa, zrow)
    ib = sb

    nch = -(-e // (NW * CH))
    if nch % 2:
        nch += 1
    e_pad = NW * nch * CH
    ia = jnp.concatenate([ia, jnp.full((e_pad - e,), zrow, jnp.int32)])
    ib = jnp.concatenate([ib, jnp.zeros((e_pad - e,), jnp.int32)])
    ia = ia.reshape(NW, nch, CH)
    ib = ib.reshape(NW, nch, CH)

    partials = _sc_bind_sum(nch)(table, ia, ib)

    scores = pl.pallas_call(
        _tc_reduce_am,
        out_shape=jax.ShapeDtypeStruct((1, am_weight.shape[0]), jnp.float32),
    )(partials, am_weight)
    return scores


# speedup vs baseline: 1.6547x; 1.6547x over previous
import jax
import jax.numpy as jnp
from jax import lax
from jax.experimental import pallas as pl
from jax.experimental.pallas import tpu as pltpu
from jax.experimental.pallas import tpu_sc as plsc

NC = 2
NS = 16
NW = NC * NS
LANES = 16
D = 256
W = 8
WP = 16
CH = 64
CHB = 7
MAIN = 13


def _sc_xor_count(nch):
    mesh = plsc.VectorSubcoreMesh(core_axis_name="c", subcore_axis_name="s")

    def body(tab, ia, ib, out, ia_v, ib_v, ba, bb, cnt_v, sem0, sem1):
        wid = lax.axis_index("s") * NC + lax.axis_index("c")
        pltpu.sync_copy(ia.at[wid], ia_v)
        pltpu.sync_copy(ib.at[wid], ib_v)
        sems = [sem0, sem1]

        def fire(c, par):
            pltpu.async_copy(tab.at[ia_v.at[c]], ba.at[par], sems[par])
            pltpu.async_copy(tab.at[ib_v.at[c]], bb.at[par], sems[par])

        def drain(c, par):
            pltpu.make_async_copy(tab.at[ia_v.at[c]], ba.at[par], sems[par]).wait()
            pltpu.make_async_copy(tab.at[ib_v.at[c]], bb.at[par], sems[par]).wait()

        fire(0, 0)

        zero = jnp.zeros((LANES,), jnp.int32)
        main0 = (zero,) * MAIN

        def pair_body(i, main):
            for par in range(2):
                c = 2 * i + par
                drain(c, par)

                @pl.when(c + 1 < nch)
                def _():
                    fire(c + 1, 1 - par)

                main = list(main)
                ch = [zero] * CHB
                for j in range(CH):
                    carry = lax.bitwise_xor(ba[par, j, :], bb[par, j, :])
                    for k in range((j + 1).bit_length()):
                        t = lax.bitwise_and(ch[k], carry)
                        ch[k] = lax.bitwise_xor(ch[k], carry)
                        carry = t
                for k in range(CHB):
                    carry = ch[k]
                    for l in range(k, MAIN):
                        t = lax.bitwise_and(main[l], carry)
                        main[l] = lax.bitwise_xor(main[l], carry)
                        carry = t
                main = tuple(main)
            return main

        main = lax.fori_loop(0, nch // 2, pair_body, main0)

        for b in range(32):
            cnt = zero
            for k in range(MAIN):
                bit = lax.bitwise_and(lax.shift_right_logical(main[k], b), 1)
                cnt = cnt + lax.shift_left(bit, k)
            cnt_v[pl.ds(b * LANES, LANES)] = cnt
        pltpu.sync_copy(cnt_v, out.at[wid])

    return pl.kernel(
        body,
        out_type=jax.ShapeDtypeStruct((NW, 32 * LANES), jnp.int32),
        mesh=mesh,
        scratch_types=[
            pltpu.VMEM((nch, CH), jnp.int32),
            pltpu.VMEM((nch, CH), jnp.int32),
            pltpu.VMEM((2, CH, WP), jnp.int32),
            pltpu.VMEM((2, CH, WP), jnp.int32),
            pltpu.VMEM((32 * LANES,), jnp.int32),
            pltpu.SemaphoreType.DMA,
            pltpu.SemaphoreType.DMA,
        ],
        compiler_params=pltpu.CompilerParams(use_tc_tiling_on_sc=False),
    )


def _tc_reduce_am(part_ref, u_ref, am_ref, out_ref):
    cnt = jnp.sum(part_ref[...], axis=0, keepdims=True)
    enc = u_ref[...] - 2.0 * cnt.astype(jnp.float32)
    out_ref[...] = lax.dot_general(
        enc, am_ref[...], (((1,), (1,)), ((), ())),
        preferred_element_type=jnp.float32,
    )


def kernel(x, edge_index, pr, ids_weight, am_weight):
    n = x.shape[0]
    d = ids_weight.shape[1]
    e = edge_index.shape[1]

    bits = (ids_weight[:n] < 0).reshape(n, W, 32).astype(jnp.int32)
    words = jnp.sum(
        jnp.left_shift(bits, jnp.arange(32, dtype=jnp.int32)), axis=-1
    )
    words = jnp.concatenate(
        [words, jnp.zeros((n, WP - W), jnp.int32)], axis=1
    )
    pr_argsort = jnp.argsort(pr)
    ptab = jnp.zeros((n, WP), jnp.int32).at[pr_argsort].set(words)

    a = jnp.minimum(edge_index[0], edge_index[1])
    b = jnp.maximum(edge_index[0], edge_index[1])
    keys = a * n + b
    ks, sa, sb = lax.sort((keys, a, b), num_keys=1)
    first = jnp.concatenate(
        [jnp.ones((1,), dtype=bool), ks[1:] != ks[:-1]]
    )
    ia = jnp.where(first, sa, 0)
    ib = jnp.where(first, sb, 0)
    u = jnp.sum(first, dtype=jnp.int32).astype(jnp.float32).reshape(1, 1)

    nch = -(-e // (NW * CH))
    if nch % 2:
        nch += 1
    e_pad = NW * nch * CH
    ia = jnp.concatenate([ia, jnp.zeros((e_pad - e,), jnp.int32)])
    ib = jnp.concatenate([ib, jnp.zeros((e_pad - e,), jnp.int32)])
    ia = ia.reshape(NW, nch, CH)
    ib = ib.reshape(NW, nch, CH)

    partials = _sc_xor_count(nch)(ptab, ia, ib)
    partials = (
        partials.reshape(NW, 32, LANES)
        .transpose(0, 2, 1)[:, :W, :]
        .reshape(NW, D)
    )

    scores = pl.pallas_call(
        _tc_reduce_am,
        out_shape=jax.ShapeDtypeStruct((1, am_weight.shape[0]), jnp.float32),
    )(partials, u, am_weight)
    return scores
